# baseline (speedup 1.0000x reference)
import functools

import jax
import jax.numpy as jnp
from jax import lax
from jax.experimental import pallas as pl
from jax.experimental.pallas import tpu as pltpu
from jax.experimental.pallas import tpu_sc as plsc

_N_ENT = 10000
_D = 128
_N_RELSLOT = 41
_REL_PAD = 48
_B = 4
_NW = 32
_CHUNK = 128
_N_CHUNKS = 40
_E_PER_W = _CHUNK * _N_CHUNKS
_E_PAD = _NW * _E_PER_W
_ACC_ROWS = 10112
_ROWS_PER_TILE = _ACC_ROWS // 16


def _rel_body(q_ref, w_ref, b_ref, out_ref):
    r = lax.dot_general(
        q_ref[...], w_ref[...], (((1,), (1,)), ((), ())),
        preferred_element_type=jnp.float32,
    )
    r = r + b_ref[...]
    out_ref[...] = r.reshape(_B, _REL_PAD, _D)


def _relation_table(query, rel_W_pad, rel_b_pad):
    return pl.pallas_call(
        _rel_body,
        out_shape=jax.ShapeDtypeStruct((_B, _REL_PAD, _D), jnp.float32),
    )(query, rel_W_pad, rel_b_pad)


_SEGS = ((0, 16), (16, 16), (32, 8))
_SEG_MAX = 16


def _sc_body(li, rel_tab, subr, relr, objr, out,
             sub_v, rel_v, obj_v, msg0, msg1, tab_v, acc, sa0, sa1, ss0, ss1):
    c_id = lax.axis_index("c")
    s_id = lax.axis_index("s")
    wid = s_id * 2 + c_id

    def _gin(b, r, msg, sem):
        return pltpu.async_copy(li.at[b].at[sub_v.at[r]], msg, sem)

    def _win(b, r, msg, sem):
        pltpu.make_async_copy(li.at[b].at[sub_v.at[r]], msg, sem).wait()

    def _mul_chunk(mk, r):
        def _grp(q, cc):
            w16 = rel_v[r, pl.ds(16 * q, 16)]
            rids = [w16[e] for e in range(16)]
            for e in range(16):
                for j in range(8):
                    mk[16 * q + e, pl.ds(j * 16, 16)] = (
                        mk[16 * q + e, pl.ds(j * 16, 16)]
                        * tab_v[rids[e], pl.ds(j * 16, 16)]
                    )
            return cc

        lax.fori_loop(0, 8, _grp, 0)

    row0 = s_id * _ROWS_PER_TILE

    def _batch(b, carry):
        pltpu.sync_copy(rel_tab.at[b], tab_v)

        def _zrow(i, cc):
            for j in range(8):
                msg0[i, pl.ds(j * 16, 16)] = jnp.zeros((16,), jnp.float32)
            return cc

        lax.fori_loop(0, 128, _zrow, 0)
        nfull = _ROWS_PER_TILE // 128
        for k in range(nfull):
            pltpu.sync_copy(msg0, acc.at[pl.ds(row0 + k * 128, 128)])
        rem = _ROWS_PER_TILE - nfull * 128
        if rem:
            pltpu.sync_copy(
                msg0.at[pl.ds(0, rem)],
                acc.at[pl.ds(row0 + nfull * 128, rem)],
            )
        plsc.subcore_barrier()

        for s0, nrows in _SEGS:
            base = wid * _N_CHUNKS + s0
            pltpu.sync_copy(subr.at[pl.ds(base, nrows)],
                            sub_v.at[pl.ds(0, nrows)])
            pltpu.sync_copy(relr.at[pl.ds(base, nrows)],
                            rel_v.at[pl.ds(0, nrows)])
            pltpu.sync_copy(objr.at[pl.ds(base, nrows)],
                            obj_v.at[pl.ds(0, nrows)])
            _gin(b, 0, msg0, sa0)
            _gin(b, 1, msg1, sa1)

            def _step(t, cc):
                for k, (mk, sk, scs, mo, so, sco) in enumerate((
                        (msg0, sa0, ss0, msg1, sa1, ss1),
                        (msg1, sa1, ss1, msg0, sa0, ss0))):
                    r = 2 * t + k
                    _win(b, r, mk, sk)
                    _mul_chunk(mk, r)
                    pltpu.async_copy(mk, acc.at[obj_v.at[r]], scs, add=True)

                    @pl.when((r >= 1) & (r + 1 <= nrows - 1))
                    def _():
                        pltpu.make_async_copy(
                            mo, acc.at[obj_v.at[r - 1]], sco).wait()
                        _gin(b, r + 1, mo, so)
                return cc

            lax.fori_loop(0, nrows // 2, _step, 0)
            pltpu.make_async_copy(msg0, acc.at[obj_v.at[0]], ss0).wait()
            pltpu.make_async_copy(msg1, acc.at[obj_v.at[0]], ss1).wait()

        plsc.subcore_barrier()
        pltpu.sync_copy(
            acc.at[pl.ds(row0, _ROWS_PER_TILE)],
            out.at[c_id, b, pl.ds(row0, _ROWS_PER_TILE)],
        )
        plsc.subcore_barrier()
        return carry

    lax.fori_loop(0, _B, _batch, 0)


def _sc_aggregate(li, rel_tab, sub2d, rel2d, obj2d):
    mesh = plsc.VectorSubcoreMesh(core_axis_name="c", subcore_axis_name="s")
    fn = functools.partial(
        pl.kernel,
        out_type=jax.ShapeDtypeStruct((2, _B, _ACC_ROWS, _D), jnp.float32),
        mesh=mesh,
        scratch_types=[
            pltpu.VMEM((_SEG_MAX, 128), jnp.int32),
            pltpu.VMEM((_SEG_MAX, 128), jnp.int32),
            pltpu.VMEM((_SEG_MAX, 128), jnp.int32),
            pltpu.VMEM((128, _D), jnp.float32),
            pltpu.VMEM((128, _D), jnp.float32),
            pltpu.VMEM((_REL_PAD, _D), jnp.float32),
            pltpu.VMEM_SHARED((_ACC_ROWS, _D), jnp.float32),
            pltpu.SemaphoreType.DMA,
            pltpu.SemaphoreType.DMA,
            pltpu.SemaphoreType.DMA,
            pltpu.SemaphoreType.DMA,
        ],
    )(_sc_body)
    return fn(li, rel_tab, sub2d, rel2d, obj2d)


def _fin_body(p0_ref, p1_ref, bd_ref, li_ref, w1_ref, w2_ref, b_ref, out_ref):
    x = p0_ref[0] + p1_ref[0] + bd_ref[0]
    li = li_ref[0]
    y = (
        lax.dot_general(x, w1_ref[...], (((1,), (1,)), ((), ())),
                        preferred_element_type=jnp.float32)
        + lax.dot_general(li, w2_ref[...], (((1,), (1,)), ((), ())),
                          preferred_element_type=jnp.float32)
        + b_ref[...]
    )
    mu = jnp.mean(y, axis=-1, keepdims=True)
    yc = y - mu
    var = jnp.mean(yc * yc, axis=-1, keepdims=True)
    z = yc * lax.rsqrt(var + 1e-5)
    out_ref[0] = jnp.maximum(z, 0.0)


def _finalize(p0, p1, boundary, layer_input, w1, w2, b2d):
    br = 1000
    grid = (_B, _N_ENT // br)
    blk = lambda: pl.BlockSpec((1, br, _D), lambda b, i: (b, i, 0))
    wblk = pl.BlockSpec((_D, _D), lambda b, i: (0, 0))
    return pl.pallas_call(
        _fin_body,
        grid=grid,
        in_specs=[
            blk(), blk(), blk(), blk(),
            wblk, wblk,
            pl.BlockSpec((1, _D), lambda b, i: (0, 0)),
        ],
        out_specs=blk(),
        out_shape=jax.ShapeDtypeStruct((_B, _N_ENT, _D), jnp.float32),
    )(p0, p1, boundary, layer_input, w1, w2, b2d)


def kernel(query, layer_input, edges, n_ent, boundary, rel_W, rel_b, Wh_W, Wh_b):
    edges = edges.astype(jnp.int32)
    sub = edges[:, 0]
    rel = edges[:, 1]
    obj = edges[:, 2]
    npad = _E_PAD - sub.shape[0]
    sub2d = jnp.concatenate([sub, jnp.zeros((npad,), jnp.int32)]).reshape(-1, _CHUNK)
    rel2d = jnp.concatenate(
        [rel, jnp.full((npad,), _N_RELSLOT, jnp.int32)]).reshape(-1, _CHUNK)
    obj2d = jnp.concatenate([obj, jnp.zeros((npad,), jnp.int32)]).reshape(-1, _CHUNK)

    w3 = rel_W.reshape(_N_RELSLOT, _D, _D)
    w3 = jnp.pad(w3, ((0, _REL_PAD - _N_RELSLOT), (0, 0), (0, 0)))
    rel_W_pad = w3.reshape(_REL_PAD * _D, _D)
    rel_b_pad = jnp.pad(rel_b, (0, (_REL_PAD - _N_RELSLOT) * _D)).reshape(1, -1)

    rel_tab = _relation_table(query, rel_W_pad, rel_b_pad)
    partials = _sc_aggregate(layer_input, rel_tab, sub2d, rel2d, obj2d)
    p0 = partials[0, :, :_N_ENT, :]
    p1 = partials[1, :, :_N_ENT, :]

    w1 = Wh_W[:, :_D]
    w2 = Wh_W[:, _D:]
    return _finalize(p0, p1, boundary, layer_input, w1, w2, Wh_b.reshape(1, _D))


# speedup vs baseline: 1.3258x; 1.3258x over previous
import functools

import jax
import jax.numpy as jnp
from jax import lax
from jax.experimental import pallas as pl
from jax.experimental.pallas import tpu as pltpu
from jax.experimental.pallas import tpu_sc as plsc

_N_ENT = 10000
_D = 128
_N_RELSLOT = 41
_REL_PAD = 48
_B = 4
_NW = 32
_CHUNK = 128
_N_CHUNKS = 40
_E_PER_W = _CHUNK * _N_CHUNKS
_E_PAD = _NW * _E_PER_W
_ACC_ROWS = 10112
_ROWS_PER_TILE = _ACC_ROWS // 16


def _rel_body(q_ref, w_ref, b_ref, out_ref):
    r = lax.dot_general(
        q_ref[...], w_ref[...], (((1,), (1,)), ((), ())),
        preferred_element_type=jnp.float32,
    )
    r = r + b_ref[...]
    out_ref[...] = r.reshape(_B, _REL_PAD, _D)


def _relation_table(query, rel_W_pad, rel_b_pad):
    return pl.pallas_call(
        _rel_body,
        out_shape=jax.ShapeDtypeStruct((_B, _REL_PAD, _D), jnp.float32),
    )(query, rel_W_pad, rel_b_pad)


_SEGS = ((0, 16), (16, 16), (32, 8))
_SEG_MAX = 16


def _sc_body(li, rel_tab, idxcat, out,
             idx_v, msg0, msg1, tab_v, acc, sa0, sa1):
    c_id = lax.axis_index("c")
    s_id = lax.axis_index("s")
    wid = s_id * 2 + c_id

    def _gin(b, r, msg, sem):
        return pltpu.async_copy(li.at[b].at[idx_v.at[r]], msg, sem)

    def _win(b, r, msg, sem):
        pltpu.make_async_copy(li.at[b].at[idx_v.at[r]], msg, sem).wait()

    def _mul_chunk(mk, r, nrows):
        def _grp(q, cc):
            w16 = idx_v[nrows + r, pl.ds(16 * q, 16)]
            for e in range(16):
                rid = w16[e]
                for j in range(8):
                    mk[16 * q + e, pl.ds(j * 16, 16)] = (
                        mk[16 * q + e, pl.ds(j * 16, 16)]
                        * tab_v[rid, pl.ds(j * 16, 16)]
                    )
            return cc

        lax.fori_loop(0, 8, _grp, 0)

    row0 = s_id * _ROWS_PER_TILE

    def _batch(b, carry):
        pltpu.sync_copy(rel_tab.at[b], tab_v)

        def _zrow(i, cc):
            for j in range(8):
                msg0[i, pl.ds(j * 16, 16)] = jnp.zeros((16,), jnp.float32)
            return cc

        lax.fori_loop(0, 128, _zrow, 0)
        nfull = _ROWS_PER_TILE // 128
        for k in range(nfull):
            pltpu.sync_copy(msg0, acc.at[pl.ds(row0 + k * 128, 128)])
        rem = _ROWS_PER_TILE - nfull * 128
        if rem:
            pltpu.sync_copy(
                msg0.at[pl.ds(0, rem)],
                acc.at[pl.ds(row0 + nfull * 128, rem)],
            )
        plsc.subcore_barrier()

        off = 0
        for s0, nrows in _SEGS:
            pltpu.sync_copy(
                idxcat.at[pl.ds(wid * 3 * _N_CHUNKS + off, 3 * nrows)],
                idx_v.at[pl.ds(0, 3 * nrows)])
            off += 3 * nrows
            _gin(b, 0, msg0, sa0)
            _gin(b, 1, msg1, sa1)

            def _step(t, cc):
                for k, (mk, sk) in enumerate(((msg0, sa0), (msg1, sa1))):
                    r = 2 * t + k
                    _win(b, r, mk, sk)
                    _mul_chunk(mk, r, nrows)
                    pltpu.sync_copy(
                        mk, acc.at[idx_v.at[2 * nrows + r]], add=True)

                    @pl.when(r < nrows - 2)
                    def _():
                        _gin(b, r + 2, mk, sk)
                return cc

            lax.fori_loop(0, nrows // 2, _step, 0)

        plsc.subcore_barrier()
        pltpu.sync_copy(
            acc.at[pl.ds(row0, _ROWS_PER_TILE)],
            out.at[c_id, b, pl.ds(row0, _ROWS_PER_TILE)],
        )
        plsc.subcore_barrier()
        return carry

    lax.fori_loop(0, _B, _batch, 0)


def _sc_aggregate(li, rel_tab, idxcat):
    mesh = plsc.VectorSubcoreMesh(core_axis_name="c", subcore_axis_name="s")
    fn = functools.partial(
        pl.kernel,
        out_type=jax.ShapeDtypeStruct((2, _B, _ACC_ROWS, _D), jnp.float32),
        mesh=mesh,
        scratch_types=[
            pltpu.VMEM((3 * _SEG_MAX, 128), jnp.int32),
            pltpu.VMEM((128, _D), jnp.float32),
            pltpu.VMEM((128, _D), jnp.float32),
            pltpu.VMEM((_REL_PAD, _D), jnp.float32),
            pltpu.VMEM_SHARED((_ACC_ROWS, _D), jnp.float32),
            pltpu.SemaphoreType.DMA,
            pltpu.SemaphoreType.DMA,
        ],
    )(_sc_body)
    return fn(li, rel_tab, idxcat)


def _fin_body(p0_ref, p1_ref, bd_ref, li_ref, w1_ref, w2_ref, b_ref, out_ref):
    x = p0_ref[0] + p1_ref[0] + bd_ref[0]
    li = li_ref[0]
    y = (
        lax.dot_general(x, w1_ref[...], (((1,), (1,)), ((), ())),
                        preferred_element_type=jnp.float32)
        + lax.dot_general(li, w2_ref[...], (((1,), (1,)), ((), ())),
                          preferred_element_type=jnp.float32)
        + b_ref[...]
    )
    mu = jnp.mean(y, axis=-1, keepdims=True)
    yc = y - mu
    var = jnp.mean(yc * yc, axis=-1, keepdims=True)
    z = yc * lax.rsqrt(var + 1e-5)
    out_ref[0] = jnp.maximum(z, 0.0)


def _finalize(p0, p1, boundary, layer_input, w1, w2, b2d):
    br = 1000
    grid = (_B, _N_ENT // br)
    blk = lambda: pl.BlockSpec((1, br, _D), lambda b, i: (b, i, 0))
    wblk = pl.BlockSpec((_D, _D), lambda b, i: (0, 0))
    return pl.pallas_call(
        _fin_body,
        grid=grid,
        in_specs=[
            blk(), blk(), blk(), blk(),
            wblk, wblk,
            pl.BlockSpec((1, _D), lambda b, i: (0, 0)),
        ],
        out_specs=blk(),
        out_shape=jax.ShapeDtypeStruct((_B, _N_ENT, _D), jnp.float32),
    )(p0, p1, boundary, layer_input, w1, w2, b2d)


def kernel(query, layer_input, edges, n_ent, boundary, rel_W, rel_b, Wh_W, Wh_b):
    edges = edges.astype(jnp.int32)
    sub = edges[:, 0]
    rel = edges[:, 1]
    obj = edges[:, 2]
    npad = _E_PAD - sub.shape[0]
    s3 = jnp.concatenate([sub, jnp.zeros((npad,), jnp.int32)]).reshape(
        _NW, _N_CHUNKS, _CHUNK)
    r3 = jnp.concatenate(
        [rel, jnp.full((npad,), _N_RELSLOT, jnp.int32)]).reshape(
        _NW, _N_CHUNKS, _CHUNK)
    o3 = jnp.concatenate([obj, jnp.zeros((npad,), jnp.int32)]).reshape(
        _NW, _N_CHUNKS, _CHUNK)
    blocks = []
    for s0, n in _SEGS:
        blocks.append(jnp.concatenate(
            [s3[:, s0:s0 + n], r3[:, s0:s0 + n], o3[:, s0:s0 + n]], axis=1))
    idxcat = jnp.concatenate(blocks, axis=1).reshape(-1, _CHUNK)

    w3 = rel_W.reshape(_N_RELSLOT, _D, _D)
    w3 = jnp.pad(w3, ((0, _REL_PAD - _N_RELSLOT), (0, 0), (0, 0)))
    rel_W_pad = w3.reshape(_REL_PAD * _D, _D)
    rel_b_pad = jnp.pad(rel_b, (0, (_REL_PAD - _N_RELSLOT) * _D)).reshape(1, -1)

    rel_tab = _relation_table(query, rel_W_pad, rel_b_pad)
    partials = _sc_aggregate(layer_input, rel_tab, idxcat)
    p0 = partials[0, :, :_N_ENT, :]
    p1 = partials[1, :, :_N_ENT, :]

    w1 = Wh_W[:, :_D]
    w2 = Wh_W[:, _D:]
    return _finalize(p0, p1, boundary, layer_input, w1, w2, Wh_b.reshape(1, _D))
